# Initial kernel scaffold; baseline (speedup 1.0000x reference)
import jax
import jax.numpy as jnp
from jax.experimental import pallas as pl


def kernel(pos, x, batch, params):
    raise NotImplementedError("write your pallas kernel here")


# speedup vs baseline: 4.4123x; 4.4123x over previous
import functools

import jax
import jax.numpy as jnp
from jax.experimental import pallas as pl

_PLANES = [32, 64, 128, 256, 512]
_STRIDE = [1, 4, 4, 4, 4]
_NSAMPLE = [8, 16, 16, 16, 16]
_NBLOCKS = [2, 3, 4, 6, 3]
_SHARE = 8
_LEVEL_N = [4096, 1024, 256, 64, 16]


def _fps_level(px, py, pz, n_valid, m, out_rows):
    rows = px.shape[0]
    iota = (jax.lax.broadcasted_iota(jnp.int32, (rows, 128), 0) * 128
            + jax.lax.broadcasted_iota(jnp.int32, (rows, 128), 1))
    valid = iota < n_valid
    dmin0 = jnp.where(valid, jnp.float32(1e10), jnp.float32(-1e30))
    oiota = (jax.lax.broadcasted_iota(jnp.int32, (out_rows, 128), 0) * 128
             + jax.lax.broadcasted_iota(jnp.int32, (out_rows, 128), 1))

    def extract(j):
        mask = iota == j
        zx = jnp.sum(jnp.where(mask, px, 0.0))
        zy = jnp.sum(jnp.where(mask, py, 0.0))
        zz = jnp.sum(jnp.where(mask, pz, 0.0))
        return zx, zy, zz

    jx, jy, jz = extract(0)
    om0 = oiota == 0
    sidx = jnp.where(om0, 0, jnp.zeros((out_rows, 128), jnp.int32))
    ox = jnp.where(om0, jx, jnp.zeros((out_rows, 128), jnp.float32))
    oy = jnp.where(om0, jy, jnp.zeros((out_rows, 128), jnp.float32))
    oz = jnp.where(om0, jz, jnp.zeros((out_rows, 128), jnp.float32))

    def body(i, c):
        dmin, sidx, ox, oy, oz, jx, jy, jz = c
        d = (px - jx) ** 2 + (py - jy) ** 2 + (pz - jz) ** 2
        dmin = jnp.minimum(dmin, d)
        mx = jnp.max(dmin)
        j = jnp.min(jnp.where(dmin == mx, iota, n_valid))
        jx, jy, jz = extract(j)
        om = oiota == i
        sidx = jnp.where(om, j, sidx)
        ox = jnp.where(om, jx, ox)
        oy = jnp.where(om, jy, oy)
        oz = jnp.where(om, jz, oz)
        return dmin, sidx, ox, oy, oz, jx, jy, jz

    c = jax.lax.fori_loop(1, m, body, (dmin0, sidx, ox, oy, oz, jx, jy, jz))
    return c[1], c[2], c[3], c[4]


def _fps_kernel(px_ref, py_ref, pz_ref,
                s2_ref, x2_ref, y2_ref, z2_ref,
                s3_ref, x3_ref, y3_ref, z3_ref,
                s4_ref, x4_ref, y4_ref, z4_ref,
                s5_ref, x5_ref, y5_ref, z5_ref):
    s2, x2, y2, z2 = _fps_level(px_ref[:], py_ref[:], pz_ref[:], 4096, 1024, 8)
    s2_ref[:], x2_ref[:], y2_ref[:], z2_ref[:] = s2, x2, y2, z2
    s3, x3, y3, z3 = _fps_level(x2, y2, z2, 1024, 256, 2)
    s3_ref[:], x3_ref[:], y3_ref[:], z3_ref[:] = s3, x3, y3, z3
    s4, x4, y4, z4 = _fps_level(x3, y3, z3, 256, 64, 1)
    s4_ref[:], x4_ref[:], y4_ref[:], z4_ref[:] = s4, x4, y4, z4
    s5, x5, y5, z5 = _fps_level(x4, y4, z4, 64, 16, 1)
    s5_ref[:], x5_ref[:], y5_ref[:], z5_ref[:] = s5, x5, y5, z5


def _run_fps(pos):
    px = pos[:, 0].reshape(32, 128)
    py = pos[:, 1].reshape(32, 128)
    pz = pos[:, 2].reshape(32, 128)
    shapes = []
    for rows in (8, 2, 1, 1):
        shapes.append(jax.ShapeDtypeStruct((rows, 128), jnp.int32))
        shapes += [jax.ShapeDtypeStruct((rows, 128), jnp.float32)] * 3
    outs = pl.pallas_call(_fps_kernel, out_shape=shapes)(px, py, pz)
    res = []
    for li, m in enumerate((1024, 256, 64, 16)):
        s = outs[4 * li].reshape(-1)[:m]
        p = jnp.stack([outs[4 * li + 1 + a].reshape(-1)[:m] for a in range(3)],
                      axis=1)
        res.append((s, p))
    return res


def _knn_body(k, nrp, want_w,
              qx_ref, qy_ref, qz_ref, rx_ref, ry_ref, rz_ref, idx_ref,
              *w_ref):
    qx, qy, qz = qx_ref[:], qy_ref[:], qz_ref[:]
    d = ((qx - rx_ref[:]) ** 2 + (qy - ry_ref[:]) ** 2
         + (qz - rz_ref[:]) ** 2)
    liota = jax.lax.broadcasted_iota(jnp.int32, d.shape, 1)
    cols_i, cols_d = [], []
    for _ in range(k):
        minv = jnp.min(d, axis=1, keepdims=True)
        j = jnp.min(jnp.where(d == minv, liota, nrp), axis=1, keepdims=True)
        cols_i.append(j)
        cols_d.append(minv)
        d = jnp.where(liota == j, jnp.float32(3.0e37), d)
    idx_ref[:] = jnp.concatenate(cols_i, axis=1)
    if want_w:
        dk = jnp.concatenate(cols_d, axis=1)
        w = 1.0 / (jnp.sqrt(jnp.maximum(dk, 1e-12)) + 1e-8)
        w_ref[0][:] = w / jnp.sum(w, axis=1, keepdims=True)


def _knn(q, r, k, want_w=False):
    nq, nr = q.shape[0], r.shape[0]
    nrp = max(128, ((nr + 127) // 128) * 128)
    if nrp > nr:
        r = jnp.concatenate(
            [r, jnp.full((nrp - nr, 3), 1e15, jnp.float32)], axis=0)
    bq = min(nq, 512)
    grid = nq // bq
    qcols = [q[:, a:a + 1] for a in range(3)]
    rrows = [r[:, a].reshape(1, nrp) for a in range(3)]
    out_shape = [jax.ShapeDtypeStruct((nq, k), jnp.int32)]
    if want_w:
        out_shape.append(jax.ShapeDtypeStruct((nq, k), jnp.float32))
    in_specs = ([pl.BlockSpec((bq, 1), lambda i: (i, 0))] * 3
                + [pl.BlockSpec((1, nrp), lambda i: (0, 0))] * 3)
    out_specs = [pl.BlockSpec((bq, k), lambda i: (i, 0))] * len(out_shape)
    res = pl.pallas_call(
        functools.partial(_knn_body, k, nrp, want_w),
        grid=(grid,), in_specs=in_specs, out_specs=out_specs,
        out_shape=out_shape)(*qcols, *rrows)
    return res if want_w else res[0]


def _relu(v):
    return jnp.maximum(v, 0.0)


def _lin(v, p):
    y = v @ p["w"]
    if "b" in p:
        y = y + p["b"]
    return y


def _bn(v, p, axes):
    m = jnp.mean(v, axis=axes, keepdims=True)
    var = jnp.var(v, axis=axes, keepdims=True)
    return (v - m) / jnp.sqrt(var + 1e-5) * p["g"] + p["b"]


def _pt_layer(pr, p, v_in, idx, nsample, share):
    n = v_in.shape[0]
    out = pr["q"]["w"].shape[1]
    xq = _lin(v_in, pr["q"])
    xk = _lin(v_in, pr["k"])
    xv = _lin(v_in, pr["v"])
    p_r = p[idx] - p[:, None, :]
    xk = xk[idx]
    xv = xv[idx]
    pe = _lin(p_r, pr["p1"])
    pe = _relu(_bn(pe, pr["pbn"], (0, 1)))
    pe = _lin(pe, pr["p2"])
    w = xk - xq[:, None, :] + pe
    w = _relu(_bn(w, pr["wbn1"], (0, 1)))
    w = _lin(w, pr["w1"])
    w = _relu(_bn(w, pr["wbn2"], (0, 1)))
    w = _lin(w, pr["w2"])
    w = jax.nn.softmax(w, axis=1)
    v = (xv + pe).reshape(n, nsample, share, out // share)
    return jnp.sum(v * w[:, :, None, :], axis=1).reshape(n, out)


def _pt_block(bp, p, v_in, idx, nsample, share):
    identity = v_in
    v = _relu(_bn(_lin(v_in, bp["l1"]), bp["bn1"], 0))
    v = _relu(_bn(_pt_layer(bp["tr"], p, v, idx, nsample, share),
                  bp["bn2"], 0))
    v = _bn(_lin(v, bp["l3"]), bp["bn3"], 0)
    return _relu(v + identity)


def _tu_head(tp, v):
    g = jnp.mean(v, axis=0, keepdims=True)
    g = _relu(_lin(g, tp["l2"]))
    xc = jnp.concatenate([v, jnp.broadcast_to(g, v.shape)], axis=1)
    return _relu(_bn(_lin(xc, tp["l1"]), tp["l1bn"], 0))


def _tu(tp, x1, x2, idx, w):
    a = _relu(_bn(_lin(x1, tp["l1"]), tp["l1bn"], 0))
    b = _relu(_bn(_lin(x2, tp["l2"]), tp["l2bn"], 0))
    return a + jnp.sum(b[idx] * w[:, :, None], axis=1)


def _forward(pos, x, params):
    fps = _run_fps(pos)
    p_levels = [pos] + [p for (_, p) in fps]
    self_idx = [_knn(p_levels[l], p_levels[l], _NSAMPLE[l]) for l in range(5)]
    td_idx = [_knn(p_levels[l + 1], p_levels[l], _NSAMPLE[l + 1])
              for l in range(4)]
    tu_geo = [_knn(p_levels[l], p_levels[l + 1], 3, want_w=True)
              for l in range(4)]

    feats = jnp.concatenate([pos, x], axis=1)
    skips = []
    cur = feats
    for li in range(5):
        ep = params["enc%d" % (li + 1)]
        td = ep["td"]
        if _STRIDE[li] == 1:
            cur = _relu(_bn(_lin(cur, td["lin"]), td["bn"], 0))
        else:
            n_p = p_levels[li]
            nidx = td_idx[li - 1]
            grouped = jnp.concatenate(
                [p_levels[li - 1][nidx] - n_p[:, None, :], cur[nidx]],
                axis=-1)
            y = _lin(grouped, td["lin"])
            y = _relu(_bn(y, td["bn"], (0, 1)))
            cur = jnp.max(y, axis=1)
        for bp in ep["blocks"]:
            cur = _pt_block(bp, p_levels[li], cur, self_idx[li],
                            _NSAMPLE[li], _SHARE)
        skips.append(cur)

    x5 = _tu_head(params["dec5"]["tu"], skips[4])
    for bp in params["dec5"]["blocks"]:
        x5 = _pt_block(bp, p_levels[4], x5, self_idx[4], _NSAMPLE[4], _SHARE)
    cur = x5
    for name, lv in zip(["dec4", "dec3", "dec2", "dec1"], [3, 2, 1, 0]):
        idx, w = tu_geo[lv]
        cur = _tu(params[name]["tu"], skips[lv], cur, idx, w)
        for bp in params[name]["blocks"]:
            cur = _pt_block(bp, p_levels[lv], cur, self_idx[lv],
                            _NSAMPLE[lv], _SHARE)
    out = _lin(cur, params["cls"]["l1"])
    out = _relu(_bn(out, params["cls"]["bn"], 0))
    return _lin(out, params["cls"]["l2"])


def kernel(pos, x, batch, params):
    return _forward(pos, x, params)
